# baseline (speedup 1.0000x reference)
import jax
import jax.numpy as jnp
from jax import lax
from jax.experimental import pallas as pl
from jax.experimental.pallas import tpu as pltpu
from jax.experimental.pallas import tpu_sc as plsc

BATCH = 4096
SLOTS = 8
KD = 64
RD = SLOTS * KD
LANES = 16
NCHUNK = KD // LANES
NC = 2
NS = 16
NW = NC * NS
UPW = BATCH // NW


def _mann_body(user_hbm, query_hbm, value_hbm, out_hbm,
               idx_v, rows_v, q_v, out_v, sem):
    wid = lax.axis_index("s") * NC + lax.axis_index("c")
    base = wid * UPW

    pltpu.sync_copy(user_hbm.at[pl.ds(base, UPW)], idx_v)
    gather = pltpu.async_copy(value_hbm.at[idx_v], rows_v, sem)
    pltpu.sync_copy(query_hbm.at[pl.ds(base, UPW)], q_v)
    gather.wait()

    def user_body(u):
        q = [q_v[u, pl.ds(c * LANES, LANES)] for c in range(NCHUNK)]
        mk = [[rows_v[u, pl.ds(s * KD + c * LANES, LANES)]
               for c in range(NCHUNK)] for s in range(SLOTS)]

        scores = []
        for s in range(SLOTS):
            acc = mk[s][0] * q[0]
            for c in range(1, NCHUNK):
                acc = acc + mk[s][c] * q[c]
            scores.append(jnp.sum(acc))

        m = scores[0]
        for s in range(1, SLOTS):
            m = jnp.maximum(m, scores[s])
        zeros = jnp.zeros((LANES,), jnp.float32)
        e = [jnp.exp(zeros + (sc - m)) for sc in scores]
        denom = e[0]
        for s in range(1, SLOTS):
            denom = denom + e[s]

        for c in range(NCHUNK):
            acc = e[0] * mk[0][c]
            for s in range(1, SLOTS):
                acc = acc + e[s] * mk[s][c]
            out_v[u, pl.ds(c * LANES, LANES)] = acc / denom

    plsc.parallel_loop(0, UPW, unroll=8)(user_body)
    pltpu.sync_copy(out_v, out_hbm.at[pl.ds(base, UPW)])


def kernel(user, query, Value):
    mesh = plsc.VectorSubcoreMesh(core_axis_name="c", subcore_axis_name="s")
    run = pl.kernel(
        _mann_body,
        out_type=jax.ShapeDtypeStruct((BATCH, KD), jnp.float32),
        mesh=mesh,
        compiler_params=pltpu.CompilerParams(needs_layout_passes=False),
        scratch_types=[
            pltpu.VMEM((UPW,), jnp.int32),
            pltpu.VMEM((UPW, RD), jnp.float32),
            pltpu.VMEM((UPW, KD), jnp.float32),
            pltpu.VMEM((UPW, KD), jnp.float32),
            pltpu.SemaphoreType.DMA,
        ],
    )
    return run(user.astype(jnp.int32), query,
               Value.reshape(Value.shape[0], RD))
